# baseline (speedup 1.0000x reference)
import functools

import jax
import jax.numpy as jnp
from jax import lax
from jax.experimental import pallas as pl
from jax.experimental.pallas import tpu as pltpu
from jax.experimental.pallas import tpu_sc as plsc

B, S, H = 16, 4096, 1024
D = 64
W = 2 * D
WP = 144


def _k3_body(g_ref, yt_ref, pt_ref, start_ref, left_ref, right_ref, wct_ref,
             out_ref):
    yt = yt_ref[...]
    g = g_ref[...]
    cols2 = lax.broadcasted_iota(jnp.int32, (B, B * WP), 1)
    row2 = lax.broadcasted_iota(jnp.int32, (B, B * WP), 0)
    w_in = cols2 - row2 * WP
    inblk = (w_in >= 0) & (w_in < WP)
    idx2 = start_ref[...] + w_in
    valid2 = inblk & (idx2 >= left_ref[...]) & (idx2 < right_ref[...])
    bias = jnp.where(valid2, 0.0, -1e30)
    pt = pt_ref[...]
    gauss = jnp.exp(-((idx2.astype(jnp.float32) - pt) ** 2) / (D * D / 2.0))
    gv = gauss * valid2.astype(jnp.float32)
    sf = lax.dot_general(yt, g, (((1,), (1,)), ((), ())),
                         preferred_element_type=jnp.float32)
    s = sf + bias
    m = jnp.max(s, axis=1, keepdims=True)
    e = jnp.exp(s - m)
    z = jnp.sum(e, axis=1, keepdims=True)
    at = (e / z) * gv
    ct = lax.dot_general(at, g, (((1,), (0,)), ((), ())),
                         preferred_element_type=jnp.float32)
    out_ref[...] = lax.dot_general(ct, wct_ref[...], (((1,), (0,)), ((), ())),
                                   preferred_element_type=jnp.float32)


def _make_sc_gather():
    info = plsc.get_sparse_core_info()
    nw = info.num_cores * info.num_subcores
    rows_total = B * WP
    b_per_w = rows_total // nw
    per_b = WP // b_per_w
    mesh = plsc.VectorSubcoreMesh(core_axis_name="c", subcore_axis_name="s")

    @functools.partial(
        pl.kernel, mesh=mesh,
        out_type=jax.ShapeDtypeStruct((rows_total, H), jnp.float32),
        scratch_types=[
            pltpu.VMEM((8,), jnp.int32),
        ],
    )
    def gather_k(enc_hbm, fs_hbm, out_hbm, fs_v):
        wid = lax.axis_index("s") * info.num_cores + lax.axis_index("c")
        base = pl.multiple_of(wid * b_per_w, 8)
        b = wid // per_b
        part = wid % per_b
        pltpu.sync_copy(fs_hbm.at[pl.ds(b * 8, 8)], fs_v)
        src = pl.multiple_of(fs_v[pl.ds(0, 1)][0] + part * b_per_w, 8)
        pltpu.sync_copy(enc_hbm.at[pl.ds(src, b_per_w)],
                        out_hbm.at[pl.ds(base, b_per_w), :])

    return gather_k


def kernel(encode_h, yt, encode_len, W_tan, w_pt, W_ct2ht):
    enc2d = encode_h.reshape(B * S, H)

    lens = encode_len.astype(jnp.float32)
    pt = jax.nn.sigmoid(jnp.tanh(yt @ W_tan) @ w_pt)[:, 0] * lens
    pti = jnp.floor(pt).astype(jnp.int32)
    left = jnp.maximum(0, pti - D)
    right = jnp.minimum(encode_len, pti + D)

    start = jnp.minimum((left // 8) * 8, S - WP)
    flat_start = start + jnp.arange(B, dtype=jnp.int32) * S

    gathered = _make_sc_gather()(enc2d, jnp.repeat(flat_start, 8))

    ht = pl.pallas_call(
        _k3_body,
        out_shape=jax.ShapeDtypeStruct((B, H), jnp.float32),
    )(gathered, yt, pt[:, None], start[:, None], left[:, None],
      right[:, None], W_ct2ht)
    return ht
